# baseline (speedup 1.0000x reference)
import functools

import jax
import jax.numpy as jnp
from jax import lax
from jax.experimental import pallas as pl
from jax.experimental.pallas import tpu as pltpu
from jax.experimental.pallas import tpu_sc as plsc

N_NODES = 10000
N_EDGES = 320000
C = 128
C_OUT = 10
NUM_GRAPHS = 64

NC = 2
NS = 16
NW = NC * NS

CHUNK = 128
E_PAD = 32 * 80 * 128
N0C = 80
N1C = 80
C0_TOTAL = NS * N0C
N_PAD = 10240
ROWS_PER_TILE = N_PAD // NS

BM = 1024


def _mp_body(h_hbm, src_hbm, dst_hbm, z2_hbm, z1_hbm, agg_hbm, deg_hbm,
             src_v, dst_v, rows_v, ones_v, agg_sh, deg_sh, *sems):
    c = lax.axis_index("c")
    s = lax.axis_index("s")
    base = jnp.where(c == 0, s * N0C, C0_TOTAL + s * N1C)
    nchunk = jnp.where(c == 0, N0C, N1C)

    r0 = s * ROWS_PER_TILE
    pltpu.sync_copy(z2_hbm.at[pl.ds(r0, ROWS_PER_TILE)],
                    agg_sh.at[pl.ds(r0, ROWS_PER_TILE)])
    pltpu.sync_copy(z1_hbm.at[pl.ds(r0, ROWS_PER_TILE)],
                    deg_sh.at[pl.ds(r0, ROWS_PER_TILE)])

    for i in range(CHUNK // 16):
        ones_v[pl.ds(i * 16, 16)] = jnp.ones((16,), jnp.float32)

    plsc.subcore_barrier()

    rbuf = (rows_v.at[0], rows_v.at[1])
    isem = tuple(sems[b] for b in range(4))
    gsem = (sems[4], sems[5])
    ssem = (sems[6], sems[7])

    def _load_idx(t, p):
        off = pl.multiple_of((base + t) * CHUNK, CHUNK)
        pltpu.async_copy(src_hbm.at[pl.ds(off, CHUNK)], src_v.at[p], isem[p])
        pltpu.async_copy(dst_hbm.at[pl.ds(off, CHUNK)], dst_v.at[p], isem[p])

    def _phase(j, p):
        t = 4 * j + p
        p2 = p % 2
        p1 = (p + 1) % 2
        p4n = (p + 1) % 4

        @pl.when((t >= 2) & (t < nchunk + 2))
        def _():
            pltpu.make_async_copy(rbuf[p2], agg_sh.at[dst_v.at[0]],
                                  ssem[p2]).wait()
            pltpu.make_async_copy(ones_v, deg_sh.at[dst_v.at[0]],
                                  ssem[p2]).wait()

        @pl.when(t < nchunk)
        def _():
            pltpu.make_async_copy(src_hbm.at[pl.ds(0, CHUNK)], src_v.at[p],
                                  isem[p]).wait()
            pltpu.make_async_copy(dst_hbm.at[pl.ds(0, CHUNK)], dst_v.at[p],
                                  isem[p]).wait()
            pltpu.async_copy(h_hbm.at[src_v.at[p]], rbuf[p2], gsem[p2])

        @pl.when(t + 1 < nchunk)
        def _():
            _load_idx(t + 1, p4n)

        @pl.when((t >= 1) & (t < nchunk + 1))
        def _():
            pltpu.make_async_copy(h_hbm.at[src_v.at[0]], rbuf[p1],
                                  gsem[p1]).wait()
            pq = (p + 3) % 4
            pltpu.async_copy(rbuf[p1], agg_sh.at[dst_v.at[pq]], ssem[p1],
                             add=True)
            pltpu.async_copy(ones_v, deg_sh.at[dst_v.at[pq]], ssem[p1],
                             add=True)

    _load_idx(0, 0)

    def body(j, carry):
        for p in range(4):
            _phase(j, p)
        return carry

    lax.fori_loop(0, (nchunk + 5) // 4, body, 0)

    plsc.subcore_barrier()

    pltpu.sync_copy(agg_sh.at[pl.ds(r0, ROWS_PER_TILE)],
                    agg_hbm.at[c, pl.ds(r0, ROWS_PER_TILE)])
    pltpu.sync_copy(deg_sh.at[pl.ds(r0, ROWS_PER_TILE)],
                    deg_hbm.at[c, pl.ds(r0, ROWS_PER_TILE)])


@functools.cache
def _get_mp_call():
    return pl.kernel(
        _mp_body,
        out_type=(
            jax.ShapeDtypeStruct((NC, N_PAD, C), jnp.float32),
            jax.ShapeDtypeStruct((NC, N_PAD), jnp.float32),
        ),
        mesh=plsc.VectorSubcoreMesh(core_axis_name="c", subcore_axis_name="s"),
        scratch_types=[
            pltpu.VMEM((4, CHUNK), jnp.int32),
            pltpu.VMEM((4, CHUNK), jnp.int32),
            pltpu.VMEM((2, CHUNK, C), jnp.float32),
            pltpu.VMEM((CHUNK,), jnp.float32),
            pltpu.VMEM_SHARED((N_PAD, C), jnp.float32),
            pltpu.VMEM_SHARED((N_PAD,), jnp.float32),
        ] + [pltpu.SemaphoreType.DMA] * 8,
    )


def _mp_call(h, src3, dst3, z2, z1):
    return _get_mp_call()(h, src3, dst3, z2, z1)


def _mm_body(x_ref, w_ref, o_ref):
    o_ref[...] = jnp.dot(x_ref[...], w_ref[...],
                         preferred_element_type=jnp.float32)


def _matmul(x, w):
    return pl.pallas_call(
        _mm_body,
        grid=(N_PAD // BM,),
        in_specs=[pl.BlockSpec((BM, C), lambda i: (i, 0)),
                  pl.BlockSpec((C, C), lambda i: (0, 0))],
        out_specs=pl.BlockSpec((BM, C), lambda i: (i, 0)),
        out_shape=jax.ShapeDtypeStruct((N_PAD, C), jnp.float32),
    )(x, w)


def _mid_body(agg_ref, h_ref, deg_ref, b_ref, w_ref, o_ref):
    a = agg_ref[0] + agg_ref[1] + h_ref[...]
    dinv = 1.0 / (deg_ref[0] + deg_ref[1] + 1.0)
    hmid = jnp.maximum(a * dinv + b_ref[...], 0.0)
    o_ref[...] = jnp.dot(hmid, w_ref[...], preferred_element_type=jnp.float32)


def _mid(agg, h, deg, b, w):
    return pl.pallas_call(
        _mid_body,
        grid=(N_PAD // BM,),
        in_specs=[pl.BlockSpec((2, BM, C), lambda i: (0, i, 0)),
                  pl.BlockSpec((BM, C), lambda i: (i, 0)),
                  pl.BlockSpec((2, BM, 1), lambda i: (0, i, 0)),
                  pl.BlockSpec((1, C), lambda i: (0, 0)),
                  pl.BlockSpec((C, C), lambda i: (0, 0))],
        out_specs=pl.BlockSpec((BM, C), lambda i: (i, 0)),
        out_shape=jax.ShapeDtypeStruct((N_PAD, C), jnp.float32),
    )(agg, h, deg, b, w)


def _tail_body(agg_ref, h_ref, deg_ref, b_ref, bidx_ref, wh_ref, bh_ref,
               o_ref, pool_ref, cnt_ref):
    i = pl.program_id(0)

    @pl.when(i == 0)
    def _():
        pool_ref[...] = jnp.zeros_like(pool_ref)
        cnt_ref[...] = jnp.zeros_like(cnt_ref)

    a = agg_ref[0] + agg_ref[1] + h_ref[...]
    dinv = 1.0 / (deg_ref[0] + deg_ref[1] + 1.0)
    h3 = a * dinv + b_ref[...]
    onehot = (bidx_ref[...] ==
              lax.broadcasted_iota(jnp.int32, (1, NUM_GRAPHS), 1)
              ).astype(jnp.float32)
    dn = (((0,), (0,)), ((), ()))
    pool_ref[...] += lax.dot_general(onehot, h3, dn,
                                     preferred_element_type=jnp.float32)
    cnt_ref[...] += lax.dot_general(onehot, jnp.ones_like(h3), dn,
                                    preferred_element_type=jnp.float32)

    @pl.when(i == pl.num_programs(0) - 1)
    def _():
        pooled = pool_ref[...] / jnp.maximum(cnt_ref[...], 1.0)
        o_ref[...] = jnp.dot(pooled, wh_ref[...],
                             preferred_element_type=jnp.float32) + bh_ref[...]


def _tail(agg, h, deg, b, bidx, wh, bh):
    return pl.pallas_call(
        _tail_body,
        grid=(N_PAD // BM,),
        in_specs=[pl.BlockSpec((2, BM, C), lambda i: (0, i, 0)),
                  pl.BlockSpec((BM, C), lambda i: (i, 0)),
                  pl.BlockSpec((2, BM, 1), lambda i: (0, i, 0)),
                  pl.BlockSpec((1, C), lambda i: (0, 0)),
                  pl.BlockSpec((BM, 1), lambda i: (i, 0)),
                  pl.BlockSpec((C, C_OUT), lambda i: (0, 0)),
                  pl.BlockSpec((1, C_OUT), lambda i: (0, 0))],
        out_specs=pl.BlockSpec((NUM_GRAPHS, C_OUT), lambda i: (0, 0)),
        out_shape=jax.ShapeDtypeStruct((NUM_GRAPHS, C_OUT), jnp.float32),
        scratch_shapes=[pltpu.VMEM((NUM_GRAPHS, C), jnp.float32),
                        pltpu.VMEM((NUM_GRAPHS, C), jnp.float32)],
    )(agg, h, deg, b, bidx, wh, bh)


def kernel(x, edge_index, batch_idx, W1, b1, W2, b2, Wh, bh):
    ei = edge_index.astype(jnp.int32)
    npad = E_PAD - N_EDGES
    src3 = jnp.concatenate([ei[0], jnp.zeros((npad,), jnp.int32)])
    pad_dst = N_NODES + jnp.arange(npad, dtype=jnp.int32) % (N_PAD - N_NODES)
    dst3 = jnp.concatenate([ei[1], pad_dst])
    z2 = jnp.zeros((N_PAD, C), jnp.float32)
    z1 = jnp.zeros((N_PAD,), jnp.float32)
    xp = jnp.concatenate([x, jnp.zeros((N_PAD - N_NODES, C), x.dtype)])
    bidx = jnp.concatenate(
        [batch_idx.astype(jnp.int32),
         jnp.full((N_PAD - N_NODES,), NUM_GRAPHS, jnp.int32)]
    ).reshape(N_PAD, 1)

    h1 = _matmul(xp, W1)
    agg1, deg = _mp_call(h1, src3, dst3, z2, z1)
    deg3 = deg.reshape(2, N_PAD, 1)
    h2 = _mid(agg1, h1, deg3, b1.reshape(1, C), W2)
    agg2, _ = _mp_call(h2, src3, dst3, z2, z1)
    out = _tail(agg2, h2, deg3, b2.reshape(1, C), bidx, Wh,
                bh.reshape(1, C_OUT))
    return out


# speedup vs baseline: 2.9489x; 2.9489x over previous
import functools

import jax
import jax.numpy as jnp
from jax import lax
from jax.experimental import pallas as pl
from jax.experimental.pallas import tpu as pltpu
from jax.experimental.pallas import tpu_sc as plsc

N_NODES = 10000
N_EDGES = 320000
C = 128
C_OUT = 10
NUM_GRAPHS = 64

NC = 2
NS = 16
NW = NC * NS

CHUNK = 128
E_PAD = 32 * 80 * 128
N0C = 80
N1C = 80
C0_TOTAL = NS * N0C
N_PAD = 10240
ROWS_PER_TILE = N_PAD // NS

BM = 1024


def _mp_body(h_hbm, src_hbm, dst_hbm, z2_hbm, z1_hbm, agg_hbm, deg_hbm,
             src_v, dst_v, rows_v, ones_v, agg_sh, deg_sh, *sems):
    c = lax.axis_index("c")
    s = lax.axis_index("s")
    base = jnp.where(c == 0, s * N0C, C0_TOTAL + s * N1C)
    nchunk = jnp.where(c == 0, N0C, N1C)

    r0 = s * ROWS_PER_TILE
    pltpu.sync_copy(z2_hbm.at[pl.ds(r0, ROWS_PER_TILE)],
                    agg_sh.at[pl.ds(r0, ROWS_PER_TILE)])
    pltpu.sync_copy(z1_hbm.at[pl.ds(r0, ROWS_PER_TILE)],
                    deg_sh.at[pl.ds(r0, ROWS_PER_TILE)])

    for i in range(CHUNK // 16):
        ones_v[pl.ds(i * 16, 16)] = jnp.ones((16,), jnp.float32)

    plsc.subcore_barrier()

    rbuf = (rows_v.at[0], rows_v.at[1])
    isem = tuple(sems[b] for b in range(4))
    gsem = (sems[4], sems[5])
    ssem = (sems[6], sems[7])

    def _load_idx(t, p):
        off = pl.multiple_of((base + t) * CHUNK, CHUNK)
        pltpu.async_copy(src_hbm.at[pl.ds(off, CHUNK)], src_v.at[p], isem[p])
        pltpu.async_copy(dst_hbm.at[pl.ds(off, CHUNK)], dst_v.at[p], isem[p])

    def _phase(j, p):
        t = 4 * j + p
        p2 = p % 2
        p1 = (p + 1) % 2
        p4n = (p + 1) % 4

        @pl.when((t >= 2) & (t < nchunk + 2))
        def _():
            pltpu.make_async_copy(rbuf[p2], agg_sh.at[dst_v.at[0]],
                                  ssem[p2]).wait()
            pltpu.make_async_copy(ones_v, deg_sh.at[dst_v.at[0]],
                                  ssem[p2]).wait()

        @pl.when(t < nchunk)
        def _():
            pltpu.make_async_copy(src_hbm.at[pl.ds(0, CHUNK)], src_v.at[p],
                                  isem[p]).wait()
            pltpu.make_async_copy(dst_hbm.at[pl.ds(0, CHUNK)], dst_v.at[p],
                                  isem[p]).wait()
            pltpu.async_copy(h_hbm.at[src_v.at[p]], rbuf[p2], gsem[p2])

        @pl.when(t + 1 < nchunk)
        def _():
            _load_idx(t + 1, p4n)

        @pl.when((t >= 1) & (t < nchunk + 1))
        def _():
            pltpu.make_async_copy(h_hbm.at[src_v.at[0]], rbuf[p1],
                                  gsem[p1]).wait()
            pq = (p + 3) % 4
            pltpu.async_copy(rbuf[p1], agg_sh.at[dst_v.at[pq]], ssem[p1],
                             add=True)
            pltpu.async_copy(ones_v, deg_sh.at[dst_v.at[pq]], ssem[p1],
                             add=True)

    _load_idx(0, 0)

    def body(j, carry):
        for p in range(4):
            _phase(j, p)
        return carry

    lax.fori_loop(0, (nchunk + 5) // 4, body, 0)

    plsc.subcore_barrier()

    pltpu.sync_copy(agg_sh.at[pl.ds(r0, ROWS_PER_TILE)],
                    agg_hbm.at[c, pl.ds(r0, ROWS_PER_TILE)])
    pltpu.sync_copy(deg_sh.at[pl.ds(r0, ROWS_PER_TILE)],
                    deg_hbm.at[c, pl.ds(r0, ROWS_PER_TILE)])


@functools.cache
def _get_mp_call():
    return pl.kernel(
        _mp_body,
        out_type=(
            jax.ShapeDtypeStruct((NC, N_PAD, C), jnp.float32),
            jax.ShapeDtypeStruct((NC, N_PAD), jnp.float32),
        ),
        mesh=plsc.VectorSubcoreMesh(core_axis_name="c", subcore_axis_name="s"),
        scratch_types=[
            pltpu.VMEM((4, CHUNK), jnp.int32),
            pltpu.VMEM((4, CHUNK), jnp.int32),
            pltpu.VMEM((2, CHUNK, C), jnp.float32),
            pltpu.VMEM((CHUNK,), jnp.float32),
            pltpu.VMEM_SHARED((N_PAD, C), jnp.float32),
            pltpu.VMEM_SHARED((N_PAD,), jnp.float32),
        ] + [pltpu.SemaphoreType.DMA] * 8,
    )


def _mp_call(h, src3, dst3, z2, z1):
    return _get_mp_call()(h, src3, dst3, z2, z1)


def _mm_body(x_ref, w_ref, o_ref):
    o_ref[...] = jnp.dot(x_ref[...], w_ref[...],
                         preferred_element_type=jnp.float32)


def _matmul(x, w):
    return pl.pallas_call(
        _mm_body,
        grid=(N_PAD // BM,),
        in_specs=[pl.BlockSpec((BM, C), lambda i: (i, 0)),
                  pl.BlockSpec((C, C), lambda i: (0, 0))],
        out_specs=pl.BlockSpec((BM, C), lambda i: (i, 0)),
        out_shape=jax.ShapeDtypeStruct((N_PAD, C), jnp.float32),
    )(x, w)


def _mid_body(agg_ref, h_ref, deg_ref, b_ref, w_ref, o_ref):
    a = agg_ref[0] + agg_ref[1] + h_ref[...]
    dinv = 1.0 / (deg_ref[0] + deg_ref[1] + 1.0)
    hmid = jnp.maximum(a * dinv + b_ref[...], 0.0)
    o_ref[...] = jnp.dot(hmid, w_ref[...], preferred_element_type=jnp.float32)


def _mid(agg, h, deg, b, w):
    return pl.pallas_call(
        _mid_body,
        grid=(N_PAD // BM,),
        in_specs=[pl.BlockSpec((2, BM, C), lambda i: (0, i, 0)),
                  pl.BlockSpec((BM, C), lambda i: (i, 0)),
                  pl.BlockSpec((2, BM, 1), lambda i: (0, i, 0)),
                  pl.BlockSpec((1, C), lambda i: (0, 0)),
                  pl.BlockSpec((C, C), lambda i: (0, 0))],
        out_specs=pl.BlockSpec((BM, C), lambda i: (i, 0)),
        out_shape=jax.ShapeDtypeStruct((N_PAD, C), jnp.float32),
    )(agg, h, deg, b, w)


def _tail_body(agg_ref, h_ref, deg_ref, b_ref, bidx_ref, wh_ref, bh_ref,
               o_ref, pool_ref, cnt_ref):
    i = pl.program_id(0)

    @pl.when(i == 0)
    def _():
        pool_ref[...] = jnp.zeros_like(pool_ref)
        cnt_ref[...] = jnp.zeros_like(cnt_ref)

    a = agg_ref[0] + agg_ref[1] + h_ref[...]
    dinv = 1.0 / (deg_ref[0] + deg_ref[1] + 1.0)
    h3 = a * dinv + b_ref[...]
    onehot = (bidx_ref[...] ==
              lax.broadcasted_iota(jnp.int32, (1, NUM_GRAPHS), 1)
              ).astype(jnp.float32)
    dn = (((0,), (0,)), ((), ()))
    pool_ref[...] += lax.dot_general(onehot, h3, dn,
                                     preferred_element_type=jnp.float32)
    cnt_ref[...] += lax.dot_general(onehot, jnp.ones_like(h3), dn,
                                    preferred_element_type=jnp.float32)

    @pl.when(i == pl.num_programs(0) - 1)
    def _():
        pooled = pool_ref[...] / jnp.maximum(cnt_ref[...], 1.0)
        o_ref[...] = jnp.dot(pooled, wh_ref[...],
                             preferred_element_type=jnp.float32) + bh_ref[...]


def _tail(agg, h, deg, b, bidx, wh, bh):
    return pl.pallas_call(
        _tail_body,
        grid=(N_PAD // BM,),
        in_specs=[pl.BlockSpec((2, BM, C), lambda i: (0, i, 0)),
                  pl.BlockSpec((BM, C), lambda i: (i, 0)),
                  pl.BlockSpec((2, BM, 1), lambda i: (0, i, 0)),
                  pl.BlockSpec((1, C), lambda i: (0, 0)),
                  pl.BlockSpec((BM, 1), lambda i: (i, 0)),
                  pl.BlockSpec((C, C_OUT), lambda i: (0, 0)),
                  pl.BlockSpec((1, C_OUT), lambda i: (0, 0))],
        out_specs=pl.BlockSpec((NUM_GRAPHS, C_OUT), lambda i: (0, 0)),
        out_shape=jax.ShapeDtypeStruct((NUM_GRAPHS, C_OUT), jnp.float32),
        scratch_shapes=[pltpu.VMEM((NUM_GRAPHS, C), jnp.float32),
                        pltpu.VMEM((NUM_GRAPHS, C), jnp.float32)],
    )(agg, h, deg, b, bidx, wh, bh)


def kernel(x, edge_index, batch_idx, W1, b1, W2, b2, Wh, bh):
    ei = edge_index.astype(jnp.int32)
    npad = E_PAD - N_EDGES
    pad_src = jnp.arange(npad, dtype=jnp.int32) % N_NODES
    src3 = jnp.concatenate([ei[0], pad_src])
    pad_dst = N_NODES + jnp.arange(npad, dtype=jnp.int32) % (N_PAD - N_NODES)
    dst3 = jnp.concatenate([ei[1], pad_dst])
    z2 = jnp.zeros((N_PAD, C), jnp.float32)
    z1 = jnp.zeros((N_PAD,), jnp.float32)
    xp = jnp.concatenate([x, jnp.zeros((N_PAD - N_NODES, C), x.dtype)])
    bidx = jnp.concatenate(
        [batch_idx.astype(jnp.int32),
         jnp.full((N_PAD - N_NODES,), NUM_GRAPHS, jnp.int32)]
    ).reshape(N_PAD, 1)

    h1 = _matmul(xp, W1)
    agg1, deg = _mp_call(h1, src3, dst3, z2, z1)
    deg3 = deg.reshape(2, N_PAD, 1)
    h2 = _mid(agg1, h1, deg3, b1.reshape(1, C), W2)
    agg2, _ = _mp_call(h2, src3, dst3, z2, z1)
    out = _tail(agg2, h2, deg3, b2.reshape(1, C), bidx, Wh,
                bh.reshape(1, C_OUT))
    return out


# speedup vs baseline: 2.9994x; 1.0171x over previous
import functools

import jax
import jax.numpy as jnp
from jax import lax
from jax.experimental import pallas as pl
from jax.experimental.pallas import tpu as pltpu
from jax.experimental.pallas import tpu_sc as plsc

N_NODES = 10000
N_EDGES = 320000
C = 128
C_OUT = 10
NUM_GRAPHS = 64

NC = 2
NS = 16
NW = NC * NS

CHUNK = 128
E_PAD = 32 * 80 * 128
N0C = 80
N1C = 80
C0_TOTAL = NS * N0C
N_PAD = 10240
ROWS_PER_TILE = N_PAD // NS

BM = 1024


def _make_mp_body(with_deg):
    def _mp_body(*refs):
        if with_deg:
            (h_hbm, src_hbm, dst_hbm, z2_hbm, z1_hbm, agg_hbm, deg_hbm,
             src_v, dst_v, rows_v, ones_v, agg_sh, deg_sh, *sems) = refs
        else:
            (h_hbm, src_hbm, dst_hbm, z2_hbm, agg_hbm,
             src_v, dst_v, rows_v, agg_sh, *sems) = refs
        c = lax.axis_index("c")
        s = lax.axis_index("s")
        base = jnp.where(c == 0, s * N0C, C0_TOTAL + s * N1C)
        nchunk = jnp.where(c == 0, N0C, N1C)

        r0 = s * ROWS_PER_TILE
        pltpu.sync_copy(z2_hbm.at[pl.ds(r0, ROWS_PER_TILE)],
                        agg_sh.at[pl.ds(r0, ROWS_PER_TILE)])
        if with_deg:
            pltpu.sync_copy(z1_hbm.at[pl.ds(r0, ROWS_PER_TILE)],
                            deg_sh.at[pl.ds(r0, ROWS_PER_TILE)])
            for i in range(CHUNK // 16):
                ones_v[pl.ds(i * 16, 16)] = jnp.ones((16,), jnp.float32)

        plsc.subcore_barrier()

        rbuf = (rows_v.at[0], rows_v.at[1])
        isem = tuple(sems[b] for b in range(4))
        gsem = (sems[4], sems[5])
        ssem = (sems[6], sems[7])

        def _load_idx(t, p):
            off = pl.multiple_of((base + t) * CHUNK, CHUNK)
            pltpu.async_copy(src_hbm.at[pl.ds(off, CHUNK)], src_v.at[p],
                             isem[p])
            pltpu.async_copy(dst_hbm.at[pl.ds(off, CHUNK)], dst_v.at[p],
                             isem[p])

        def _phase(j, p):
            t = 4 * j + p
            p2 = p % 2
            p1 = (p + 1) % 2
            p4n = (p + 1) % 4

            @pl.when((t >= 2) & (t < nchunk + 2))
            def _():
                pltpu.make_async_copy(rbuf[p2], agg_sh.at[dst_v.at[0]],
                                      ssem[p2]).wait()
                if with_deg:
                    pltpu.make_async_copy(ones_v, deg_sh.at[dst_v.at[0]],
                                          ssem[p2]).wait()

            @pl.when(t < nchunk)
            def _():
                pltpu.make_async_copy(src_hbm.at[pl.ds(0, CHUNK)],
                                      src_v.at[p], isem[p]).wait()
                pltpu.make_async_copy(dst_hbm.at[pl.ds(0, CHUNK)],
                                      dst_v.at[p], isem[p]).wait()
                pltpu.async_copy(h_hbm.at[src_v.at[p]], rbuf[p2], gsem[p2])

            @pl.when(t + 1 < nchunk)
            def _():
                _load_idx(t + 1, p4n)

            @pl.when((t >= 1) & (t < nchunk + 1))
            def _():
                pltpu.make_async_copy(h_hbm.at[src_v.at[0]], rbuf[p1],
                                      gsem[p1]).wait()
                pq = (p + 3) % 4
                pltpu.async_copy(rbuf[p1], agg_sh.at[dst_v.at[pq]],
                                 ssem[p1], add=True)
                if with_deg:
                    pltpu.async_copy(ones_v, deg_sh.at[dst_v.at[pq]],
                                     ssem[p1], add=True)

        _load_idx(0, 0)

        def body(j, carry):
            for p in range(4):
                _phase(j, p)
            return carry

        lax.fori_loop(0, (nchunk + 5) // 4, body, 0)

        plsc.subcore_barrier()

        pltpu.sync_copy(agg_sh.at[pl.ds(r0, ROWS_PER_TILE)],
                        agg_hbm.at[c, pl.ds(r0, ROWS_PER_TILE)])
        if with_deg:
            pltpu.sync_copy(deg_sh.at[pl.ds(r0, ROWS_PER_TILE)],
                            deg_hbm.at[c, pl.ds(r0, ROWS_PER_TILE)])

    return _mp_body


@functools.cache
def _get_mp_call(with_deg):
    if with_deg:
        out_type = (
            jax.ShapeDtypeStruct((NC, N_PAD, C), jnp.float32),
            jax.ShapeDtypeStruct((NC, N_PAD), jnp.float32),
        )
        scratch = [
            pltpu.VMEM((4, CHUNK), jnp.int32),
            pltpu.VMEM((4, CHUNK), jnp.int32),
            pltpu.VMEM((2, CHUNK, C), jnp.float32),
            pltpu.VMEM((CHUNK,), jnp.float32),
            pltpu.VMEM_SHARED((N_PAD, C), jnp.float32),
            pltpu.VMEM_SHARED((N_PAD,), jnp.float32),
        ]
    else:
        out_type = (jax.ShapeDtypeStruct((NC, N_PAD, C), jnp.float32),)
        scratch = [
            pltpu.VMEM((4, CHUNK), jnp.int32),
            pltpu.VMEM((4, CHUNK), jnp.int32),
            pltpu.VMEM((2, CHUNK, C), jnp.float32),
            pltpu.VMEM_SHARED((N_PAD, C), jnp.float32),
        ]
    return pl.kernel(
        _make_mp_body(with_deg),
        out_type=out_type,
        mesh=plsc.VectorSubcoreMesh(core_axis_name="c", subcore_axis_name="s"),
        scratch_types=scratch + [pltpu.SemaphoreType.DMA] * 8,
    )


def _mp_call(h, src3, dst3, z2, z1):
    return _get_mp_call(True)(h, src3, dst3, z2, z1)


def _mp_call_nodeg(h, src3, dst3, z2):
    return _get_mp_call(False)(h, src3, dst3, z2)[0]


def _mm_body(x_ref, w_ref, o_ref):
    o_ref[...] = jnp.dot(x_ref[...], w_ref[...],
                         preferred_element_type=jnp.float32)


def _matmul(x, w):
    return pl.pallas_call(
        _mm_body,
        grid=(N_PAD // BM,),
        in_specs=[pl.BlockSpec((BM, C), lambda i: (i, 0)),
                  pl.BlockSpec((C, C), lambda i: (0, 0))],
        out_specs=pl.BlockSpec((BM, C), lambda i: (i, 0)),
        out_shape=jax.ShapeDtypeStruct((N_PAD, C), jnp.float32),
    )(x, w)


def _mid_body(agg_ref, h_ref, deg_ref, b_ref, w_ref, o_ref):
    a = agg_ref[0] + agg_ref[1] + h_ref[...]
    dinv = 1.0 / (deg_ref[0] + deg_ref[1] + 1.0)
    hmid = jnp.maximum(a * dinv + b_ref[...], 0.0)
    o_ref[...] = jnp.dot(hmid, w_ref[...], preferred_element_type=jnp.float32)


def _mid(agg, h, deg, b, w):
    return pl.pallas_call(
        _mid_body,
        grid=(N_PAD // BM,),
        in_specs=[pl.BlockSpec((2, BM, C), lambda i: (0, i, 0)),
                  pl.BlockSpec((BM, C), lambda i: (i, 0)),
                  pl.BlockSpec((2, BM, 1), lambda i: (0, i, 0)),
                  pl.BlockSpec((1, C), lambda i: (0, 0)),
                  pl.BlockSpec((C, C), lambda i: (0, 0))],
        out_specs=pl.BlockSpec((BM, C), lambda i: (i, 0)),
        out_shape=jax.ShapeDtypeStruct((N_PAD, C), jnp.float32),
    )(agg, h, deg, b, w)


def _tail_body(agg_ref, h_ref, deg_ref, b_ref, bidx_ref, wh_ref, bh_ref,
               o_ref, pool_ref, cnt_ref):
    i = pl.program_id(0)

    @pl.when(i == 0)
    def _():
        pool_ref[...] = jnp.zeros_like(pool_ref)
        cnt_ref[...] = jnp.zeros_like(cnt_ref)

    a = agg_ref[0] + agg_ref[1] + h_ref[...]
    dinv = 1.0 / (deg_ref[0] + deg_ref[1] + 1.0)
    h3 = a * dinv + b_ref[...]
    onehot = (bidx_ref[...] ==
              lax.broadcasted_iota(jnp.int32, (1, NUM_GRAPHS), 1)
              ).astype(jnp.float32)
    dn = (((0,), (0,)), ((), ()))
    pool_ref[...] += lax.dot_general(onehot, h3, dn,
                                     preferred_element_type=jnp.float32)
    cnt_ref[...] += lax.dot_general(onehot, jnp.ones_like(h3), dn,
                                    preferred_element_type=jnp.float32)

    @pl.when(i == pl.num_programs(0) - 1)
    def _():
        pooled = pool_ref[...] / jnp.maximum(cnt_ref[...], 1.0)
        o_ref[...] = jnp.dot(pooled, wh_ref[...],
                             preferred_element_type=jnp.float32) + bh_ref[...]


def _tail(agg, h, deg, b, bidx, wh, bh):
    return pl.pallas_call(
        _tail_body,
        grid=(N_PAD // BM,),
        in_specs=[pl.BlockSpec((2, BM, C), lambda i: (0, i, 0)),
                  pl.BlockSpec((BM, C), lambda i: (i, 0)),
                  pl.BlockSpec((2, BM, 1), lambda i: (0, i, 0)),
                  pl.BlockSpec((1, C), lambda i: (0, 0)),
                  pl.BlockSpec((BM, 1), lambda i: (i, 0)),
                  pl.BlockSpec((C, C_OUT), lambda i: (0, 0)),
                  pl.BlockSpec((1, C_OUT), lambda i: (0, 0))],
        out_specs=pl.BlockSpec((NUM_GRAPHS, C_OUT), lambda i: (0, 0)),
        out_shape=jax.ShapeDtypeStruct((NUM_GRAPHS, C_OUT), jnp.float32),
        scratch_shapes=[pltpu.VMEM((NUM_GRAPHS, C), jnp.float32),
                        pltpu.VMEM((NUM_GRAPHS, C), jnp.float32)],
    )(agg, h, deg, b, bidx, wh, bh)


def kernel(x, edge_index, batch_idx, W1, b1, W2, b2, Wh, bh):
    ei = edge_index.astype(jnp.int32)
    npad = E_PAD - N_EDGES
    pad_src = jnp.arange(npad, dtype=jnp.int32) % N_NODES
    src3 = jnp.concatenate([ei[0], pad_src])
    pad_dst = N_NODES + jnp.arange(npad, dtype=jnp.int32) % (N_PAD - N_NODES)
    dst3 = jnp.concatenate([ei[1], pad_dst])
    z2 = jnp.zeros((N_PAD, C), jnp.float32)
    z1 = jnp.zeros((N_PAD,), jnp.float32)
    xp = jnp.concatenate([x, jnp.zeros((N_PAD - N_NODES, C), x.dtype)])
    bidx = jnp.concatenate(
        [batch_idx.astype(jnp.int32),
         jnp.full((N_PAD - N_NODES,), NUM_GRAPHS, jnp.int32)]
    ).reshape(N_PAD, 1)

    h1 = _matmul(xp, W1)
    agg1, deg = _mp_call(h1, src3, dst3, z2, z1)
    deg3 = deg.reshape(2, N_PAD, 1)
    h2 = _mid(agg1, h1, deg3, b1.reshape(1, C), W2)
    agg2 = _mp_call_nodeg(h2, src3, dst3, z2)
    out = _tail(agg2, h2, deg3, b2.reshape(1, C), bidx, Wh,
                bh.reshape(1, C_OUT))
    return out


# speedup vs baseline: 3.0166x; 1.0058x over previous
import functools

import jax
import jax.numpy as jnp
from jax import lax
from jax.experimental import pallas as pl
from jax.experimental.pallas import tpu as pltpu
from jax.experimental.pallas import tpu_sc as plsc

N_NODES = 10000
N_EDGES = 320000
C = 128
C_OUT = 10
NUM_GRAPHS = 64

NC = 2
NS = 16
NW = NC * NS

CHUNK = 128
NCHUNKS = N_EDGES // CHUNK
NCK_LO = NCHUNKS // NW
NCK_EXTRA = NCHUNKS - NW * NCK_LO
N_PAD = 10240
ROWS_PER_TILE = N_PAD // NS

BM = 1024


def _make_mp_body(with_deg):
    def _mp_body(*refs):
        if with_deg:
            (h_hbm, src_hbm, dst_hbm, z2_hbm, z1_hbm, agg_hbm, deg_hbm,
             src_v, dst_v, rows_v, ones_v, agg_sh, deg_sh, *sems) = refs
        else:
            (h_hbm, src_hbm, dst_hbm, z2_hbm, agg_hbm,
             src_v, dst_v, rows_v, agg_sh, *sems) = refs
        c = lax.axis_index("c")
        s = lax.axis_index("s")
        wid = s * NC + c
        base = wid * NCK_LO + jnp.minimum(wid, NCK_EXTRA)
        nchunk = NCK_LO + jnp.where(wid < NCK_EXTRA, 1, 0)

        r0 = s * ROWS_PER_TILE
        pltpu.sync_copy(z2_hbm, agg_sh.at[pl.ds(r0, ROWS_PER_TILE)])
        if with_deg:
            pltpu.sync_copy(z1_hbm, deg_sh.at[pl.ds(r0, ROWS_PER_TILE)])
            for i in range(CHUNK // 16):
                ones_v[pl.ds(i * 16, 16)] = jnp.ones((16,), jnp.float32)

        plsc.subcore_barrier()

        rbuf = (rows_v.at[0], rows_v.at[1])
        isem = tuple(sems[b] for b in range(4))
        gsem = (sems[4], sems[5])
        ssem = (sems[6], sems[7])

        def _load_idx(t, p):
            off = pl.multiple_of((base + t) * CHUNK, CHUNK)
            pltpu.async_copy(src_hbm.at[pl.ds(off, CHUNK)], src_v.at[p],
                             isem[p])
            pltpu.async_copy(dst_hbm.at[pl.ds(off, CHUNK)], dst_v.at[p],
                             isem[p])

        def _phase(j, p):
            t = 4 * j + p
            p2 = p % 2
            p1 = (p + 1) % 2
            p4n = (p + 1) % 4

            @pl.when((t >= 2) & (t < nchunk + 2))
            def _():
                pltpu.make_async_copy(rbuf[p2], agg_sh.at[dst_v.at[0]],
                                      ssem[p2]).wait()
                if with_deg:
                    pltpu.make_async_copy(ones_v, deg_sh.at[dst_v.at[0]],
                                          ssem[p2]).wait()

            @pl.when(t < nchunk)
            def _():
                pltpu.make_async_copy(src_hbm.at[pl.ds(0, CHUNK)],
                                      src_v.at[p], isem[p]).wait()
                pltpu.make_async_copy(dst_hbm.at[pl.ds(0, CHUNK)],
                                      dst_v.at[p], isem[p]).wait()
                pltpu.async_copy(h_hbm.at[src_v.at[p]], rbuf[p2], gsem[p2])

            @pl.when(t + 1 < nchunk)
            def _():
                _load_idx(t + 1, p4n)

            @pl.when((t >= 1) & (t < nchunk + 1))
            def _():
                pltpu.make_async_copy(h_hbm.at[src_v.at[0]], rbuf[p1],
                                      gsem[p1]).wait()
                pq = (p + 3) % 4
                pltpu.async_copy(rbuf[p1], agg_sh.at[dst_v.at[pq]],
                                 ssem[p1], add=True)
                if with_deg:
                    pltpu.async_copy(ones_v, deg_sh.at[dst_v.at[pq]],
                                     ssem[p1], add=True)

        _load_idx(0, 0)

        def body(j, carry):
            for p in range(4):
                _phase(j, p)
            return carry

        lax.fori_loop(0, (nchunk + 5) // 4, body, 0)

        plsc.subcore_barrier()

        pltpu.sync_copy(agg_sh.at[pl.ds(r0, ROWS_PER_TILE)],
                        agg_hbm.at[c, pl.ds(r0, ROWS_PER_TILE)])
        if with_deg:
            pltpu.sync_copy(deg_sh.at[pl.ds(r0, ROWS_PER_TILE)],
                            deg_hbm.at[c, pl.ds(r0, ROWS_PER_TILE)])

    return _mp_body


@functools.cache
def _get_mp_call(with_deg):
    if with_deg:
        out_type = (
            jax.ShapeDtypeStruct((NC, N_PAD, C), jnp.float32),
            jax.ShapeDtypeStruct((NC, N_PAD), jnp.float32),
        )
        scratch = [
            pltpu.VMEM((4, CHUNK), jnp.int32),
            pltpu.VMEM((4, CHUNK), jnp.int32),
            pltpu.VMEM((2, CHUNK, C), jnp.float32),
            pltpu.VMEM((CHUNK,), jnp.float32),
            pltpu.VMEM_SHARED((N_PAD, C), jnp.float32),
            pltpu.VMEM_SHARED((N_PAD,), jnp.float32),
        ]
    else:
        out_type = (jax.ShapeDtypeStruct((NC, N_PAD, C), jnp.float32),)
        scratch = [
            pltpu.VMEM((4, CHUNK), jnp.int32),
            pltpu.VMEM((4, CHUNK), jnp.int32),
            pltpu.VMEM((2, CHUNK, C), jnp.float32),
            pltpu.VMEM_SHARED((N_PAD, C), jnp.float32),
        ]
    return pl.kernel(
        _make_mp_body(with_deg),
        out_type=out_type,
        mesh=plsc.VectorSubcoreMesh(core_axis_name="c", subcore_axis_name="s"),
        scratch_types=scratch + [pltpu.SemaphoreType.DMA] * 8,
    )


def _mp_call(h, src3, dst3, z2, z1):
    return _get_mp_call(True)(h, src3, dst3, z2, z1)


def _mp_call_nodeg(h, src3, dst3, z2):
    return _get_mp_call(False)(h, src3, dst3, z2)[0]


def _mm_body(x_ref, w_ref, o_ref):
    o_ref[...] = jnp.dot(x_ref[...], w_ref[...],
                         preferred_element_type=jnp.float32)


def _matmul(x, w):
    return pl.pallas_call(
        _mm_body,
        grid=(N_PAD // BM,),
        in_specs=[pl.BlockSpec((BM, C), lambda i: (i, 0)),
                  pl.BlockSpec((C, C), lambda i: (0, 0))],
        out_specs=pl.BlockSpec((BM, C), lambda i: (i, 0)),
        out_shape=jax.ShapeDtypeStruct((N_PAD, C), jnp.float32),
    )(x, w)


def _mid_body(agg_ref, h_ref, deg_ref, b_ref, w_ref, o_ref):
    a = agg_ref[0] + agg_ref[1] + h_ref[...]
    dinv = 1.0 / (deg_ref[0] + deg_ref[1] + 1.0)
    hmid = jnp.maximum(a * dinv + b_ref[...], 0.0)
    o_ref[...] = jnp.dot(hmid, w_ref[...], preferred_element_type=jnp.float32)


def _mid(agg, h, deg, b, w):
    return pl.pallas_call(
        _mid_body,
        grid=(N_PAD // BM,),
        in_specs=[pl.BlockSpec((2, BM, C), lambda i: (0, i, 0)),
                  pl.BlockSpec((BM, C), lambda i: (i, 0)),
                  pl.BlockSpec((2, BM, 1), lambda i: (0, i, 0)),
                  pl.BlockSpec((1, C), lambda i: (0, 0)),
                  pl.BlockSpec((C, C), lambda i: (0, 0))],
        out_specs=pl.BlockSpec((BM, C), lambda i: (i, 0)),
        out_shape=jax.ShapeDtypeStruct((N_PAD, C), jnp.float32),
    )(agg, h, deg, b, w)


def _tail_body(agg_ref, h_ref, deg_ref, b_ref, bidx_ref, wh_ref, bh_ref,
               o_ref, pool_ref, cnt_ref):
    i = pl.program_id(0)

    @pl.when(i == 0)
    def _():
        pool_ref[...] = jnp.zeros_like(pool_ref)
        cnt_ref[...] = jnp.zeros_like(cnt_ref)

    a = agg_ref[0] + agg_ref[1] + h_ref[...]
    dinv = 1.0 / (deg_ref[0] + deg_ref[1] + 1.0)
    h3 = a * dinv + b_ref[...]
    rows = i * BM + lax.broadcasted_iota(jnp.int32, (BM, 1), 0)
    h3 = jnp.where(rows < N_NODES, h3, 0.0)
    onehot = (bidx_ref[...] ==
              lax.broadcasted_iota(jnp.int32, (1, NUM_GRAPHS), 1)
              ).astype(jnp.float32)
    dn = (((0,), (0,)), ((), ()))
    pool_ref[...] += lax.dot_general(onehot, h3, dn,
                                     preferred_element_type=jnp.float32)
    cnt_ref[...] += lax.dot_general(onehot, jnp.ones_like(h3), dn,
                                    preferred_element_type=jnp.float32)

    @pl.when(i == pl.num_programs(0) - 1)
    def _():
        pooled = pool_ref[...] / jnp.maximum(cnt_ref[...], 1.0)
        o_ref[...] = jnp.dot(pooled, wh_ref[...],
                             preferred_element_type=jnp.float32) + bh_ref[...]


def _tail(agg, h, deg, b, bidx, wh, bh):
    return pl.pallas_call(
        _tail_body,
        grid=(N_PAD // BM,),
        in_specs=[pl.BlockSpec((2, BM, C), lambda i: (0, i, 0)),
                  pl.BlockSpec((BM, C), lambda i: (i, 0)),
                  pl.BlockSpec((2, BM, 1), lambda i: (0, i, 0)),
                  pl.BlockSpec((1, C), lambda i: (0, 0)),
                  pl.BlockSpec((BM, 1), lambda i: (i, 0)),
                  pl.BlockSpec((C, C_OUT), lambda i: (0, 0)),
                  pl.BlockSpec((1, C_OUT), lambda i: (0, 0))],
        out_specs=pl.BlockSpec((NUM_GRAPHS, C_OUT), lambda i: (0, 0)),
        out_shape=jax.ShapeDtypeStruct((NUM_GRAPHS, C_OUT), jnp.float32),
        scratch_shapes=[pltpu.VMEM((NUM_GRAPHS, C), jnp.float32),
                        pltpu.VMEM((NUM_GRAPHS, C), jnp.float32)],
    )(agg, h, deg, b, bidx, wh, bh)


def kernel(x, edge_index, batch_idx, W1, b1, W2, b2, Wh, bh):
    ei = edge_index.astype(jnp.int32)
    src3 = ei[0]
    dst3 = ei[1]
    z2 = jnp.zeros((ROWS_PER_TILE, C), jnp.float32)
    z1 = jnp.zeros((ROWS_PER_TILE,), jnp.float32)
    bidx = jnp.concatenate(
        [batch_idx.astype(jnp.int32),
         jnp.full((N_PAD - N_NODES,), NUM_GRAPHS, jnp.int32)]
    ).reshape(N_PAD, 1)

    h1 = _matmul(x, W1)
    agg1, deg = _mp_call(h1, src3, dst3, z2, z1)
    deg3 = deg.reshape(2, N_PAD, 1)
    h2 = _mid(agg1, h1, deg3, b1.reshape(1, C), W2)
    agg2 = _mp_call_nodeg(h2, src3, dst3, z2)
    out = _tail(agg2, h2, deg3, b2.reshape(1, C), bidx, Wh,
                bh.reshape(1, C_OUT))
    return out


# speedup vs baseline: 3.1829x; 1.0551x over previous
import functools

import jax
import jax.numpy as jnp
from jax import lax
from jax.experimental import pallas as pl
from jax.experimental.pallas import tpu as pltpu
from jax.experimental.pallas import tpu_sc as plsc

N_NODES = 10000
N_EDGES = 320000
C = 128
C_OUT = 10
NUM_GRAPHS = 64

NC = 2
NS = 16
NW = NC * NS

CHUNK = 128
NCHUNKS = N_EDGES // CHUNK
NCK_LO = NCHUNKS // NW
NCK_EXTRA = NCHUNKS - NW * NCK_LO
N_PAD = 10240
ROWS_PER_TILE = N_PAD // NS

BM = 1024


def _make_mp_body(with_deg):
    def _mp_body(*refs):
        if with_deg:
            (h_hbm, ei_hbm, z2_hbm, z1_hbm, agg_hbm, deg_hbm,
             src_v, dst_v, rows_v, ones_v, agg_sh, deg_sh, *sems) = refs
        else:
            (h_hbm, ei_hbm, z2_hbm, agg_hbm,
             src_v, dst_v, rows_v, agg_sh, *sems) = refs
        c = lax.axis_index("c")
        s = lax.axis_index("s")
        wid = s * NC + c
        base = wid * NCK_LO + jnp.minimum(wid, NCK_EXTRA)
        nchunk = NCK_LO + jnp.where(wid < NCK_EXTRA, 1, 0)

        r0 = s * ROWS_PER_TILE
        pltpu.sync_copy(z2_hbm, agg_sh.at[pl.ds(r0, ROWS_PER_TILE)])
        if with_deg:
            pltpu.sync_copy(z1_hbm, deg_sh.at[pl.ds(r0, ROWS_PER_TILE)])
            for i in range(CHUNK // 16):
                ones_v[pl.ds(i * 16, 16)] = jnp.ones((16,), jnp.float32)

        plsc.subcore_barrier()

        rbuf = (rows_v.at[0], rows_v.at[1])
        isem = tuple(sems[b] for b in range(4))
        gsem = (sems[4], sems[5])
        ssem = (sems[6], sems[7])

        def _load_idx(t, p):
            off = pl.multiple_of((base + t) * CHUNK, CHUNK)
            pltpu.async_copy(ei_hbm.at[0, pl.ds(off, CHUNK)], src_v.at[p],
                             isem[p])
            pltpu.async_copy(ei_hbm.at[1, pl.ds(off, CHUNK)], dst_v.at[p],
                             isem[p])

        def _phase(j, p):
            t = 4 * j + p
            p2 = p % 2
            p1 = (p + 1) % 2
            p4n = (p + 1) % 4

            @pl.when((t >= 2) & (t < nchunk + 2))
            def _():
                pltpu.make_async_copy(rbuf[p2], agg_sh.at[dst_v.at[0]],
                                      ssem[p2]).wait()
                if with_deg:
                    pltpu.make_async_copy(ones_v, deg_sh.at[dst_v.at[0]],
                                          ssem[p2]).wait()

            @pl.when(t < nchunk)
            def _():
                pltpu.make_async_copy(ei_hbm.at[0, pl.ds(0, CHUNK)],
                                      src_v.at[p], isem[p]).wait()
                pltpu.make_async_copy(ei_hbm.at[1, pl.ds(0, CHUNK)],
                                      dst_v.at[p], isem[p]).wait()
                pltpu.async_copy(h_hbm.at[src_v.at[p]], rbuf[p2], gsem[p2])

            @pl.when(t + 1 < nchunk)
            def _():
                _load_idx(t + 1, p4n)

            @pl.when((t >= 1) & (t < nchunk + 1))
            def _():
                pltpu.make_async_copy(h_hbm.at[src_v.at[0]], rbuf[p1],
                                      gsem[p1]).wait()
                pq = (p + 3) % 4
                pltpu.async_copy(rbuf[p1], agg_sh.at[dst_v.at[pq]],
                                 ssem[p1], add=True)
                if with_deg:
                    pltpu.async_copy(ones_v, deg_sh.at[dst_v.at[pq]],
                                     ssem[p1], add=True)

        _load_idx(0, 0)

        def body(j, carry):
            for p in range(4):
                _phase(j, p)
            return carry

        lax.fori_loop(0, (nchunk + 5) // 4, body, 0)

        plsc.subcore_barrier()

        pltpu.sync_copy(agg_sh.at[pl.ds(r0, ROWS_PER_TILE)],
                        agg_hbm.at[c, pl.ds(r0, ROWS_PER_TILE)])
        if with_deg:
            pltpu.sync_copy(deg_sh.at[pl.ds(r0, ROWS_PER_TILE)],
                            deg_hbm.at[c, pl.ds(r0, ROWS_PER_TILE)])

    return _mp_body


@functools.cache
def _get_mp_call(with_deg):
    if with_deg:
        out_type = (
            jax.ShapeDtypeStruct((NC, N_PAD, C), jnp.float32),
            jax.ShapeDtypeStruct((NC, N_PAD), jnp.float32),
        )
        scratch = [
            pltpu.VMEM((4, CHUNK), jnp.int32),
            pltpu.VMEM((4, CHUNK), jnp.int32),
            pltpu.VMEM((2, CHUNK, C), jnp.float32),
            pltpu.VMEM((CHUNK,), jnp.float32),
            pltpu.VMEM_SHARED((N_PAD, C), jnp.float32),
            pltpu.VMEM_SHARED((N_PAD,), jnp.float32),
        ]
    else:
        out_type = (jax.ShapeDtypeStruct((NC, N_PAD, C), jnp.float32),)
        scratch = [
            pltpu.VMEM((4, CHUNK), jnp.int32),
            pltpu.VMEM((4, CHUNK), jnp.int32),
            pltpu.VMEM((2, CHUNK, C), jnp.float32),
            pltpu.VMEM_SHARED((N_PAD, C), jnp.float32),
        ]
    return pl.kernel(
        _make_mp_body(with_deg),
        out_type=out_type,
        mesh=plsc.VectorSubcoreMesh(core_axis_name="c", subcore_axis_name="s"),
        scratch_types=scratch + [pltpu.SemaphoreType.DMA] * 8,
    )


def _mp_call(h, ei, z2, z1):
    return _get_mp_call(True)(h, ei, z2, z1)


def _mp_call_nodeg(h, ei, z2):
    return _get_mp_call(False)(h, ei, z2)[0]


def _mm_body(x_ref, w_ref, o_ref):
    o_ref[...] = jnp.dot(x_ref[...], w_ref[...],
                         preferred_element_type=jnp.float32)


def _matmul(x, w):
    return pl.pallas_call(
        _mm_body,
        grid=(N_PAD // BM,),
        in_specs=[pl.BlockSpec((BM, C), lambda i: (i, 0)),
                  pl.BlockSpec((C, C), lambda i: (0, 0))],
        out_specs=pl.BlockSpec((BM, C), lambda i: (i, 0)),
        out_shape=jax.ShapeDtypeStruct((N_PAD, C), jnp.float32),
    )(x, w)


def _mid_body(agg_ref, h_ref, deg_ref, b_ref, w_ref, o_ref):
    a = agg_ref[0] + agg_ref[1] + h_ref[...]
    dinv = 1.0 / (deg_ref[0] + deg_ref[1] + 1.0)
    hmid = jnp.maximum(a * dinv + b_ref[...], 0.0)
    o_ref[...] = jnp.dot(hmid, w_ref[...], preferred_element_type=jnp.float32)


def _mid(agg, h, deg, b, w):
    return pl.pallas_call(
        _mid_body,
        grid=(N_PAD // BM,),
        in_specs=[pl.BlockSpec((2, BM, C), lambda i: (0, i, 0)),
                  pl.BlockSpec((BM, C), lambda i: (i, 0)),
                  pl.BlockSpec((2, BM, 1), lambda i: (0, i, 0)),
                  pl.BlockSpec((1, C), lambda i: (0, 0)),
                  pl.BlockSpec((C, C), lambda i: (0, 0))],
        out_specs=pl.BlockSpec((BM, C), lambda i: (i, 0)),
        out_shape=jax.ShapeDtypeStruct((N_PAD, C), jnp.float32),
    )(agg, h, deg, b, w)


def _cnt_body(bidx_ref, o_ref, cnt_ref):
    i = pl.program_id(0)

    @pl.when(i == 0)
    def _():
        cnt_ref[...] = jnp.zeros_like(cnt_ref)

    onehot = (bidx_ref[...] ==
              lax.broadcasted_iota(jnp.int32, (1, NUM_GRAPHS), 1)
              ).astype(jnp.float32)
    dn = (((0,), (0,)), ((), ()))
    cnt_ref[...] += lax.dot_general(
        onehot, jnp.ones((BM, C), jnp.float32), dn,
        preferred_element_type=jnp.float32)

    @pl.when(i == pl.num_programs(0) - 1)
    def _():
        o_ref[...] = 1.0 / jnp.maximum(cnt_ref[...], 1.0)


def _cnt(bidx):
    return pl.pallas_call(
        _cnt_body,
        grid=(N_PAD // BM,),
        in_specs=[pl.BlockSpec((BM, 1), lambda i: (i, 0))],
        out_specs=pl.BlockSpec((NUM_GRAPHS, C), lambda i: (0, 0)),
        out_shape=jax.ShapeDtypeStruct((NUM_GRAPHS, C), jnp.float32),
        scratch_shapes=[pltpu.VMEM((NUM_GRAPHS, C), jnp.float32)],
    )(bidx)


def _tail_body(agg_ref, h_ref, deg_ref, b_ref, bidx_ref, cinv_ref, wh_ref,
               bh_ref, o_ref, pool_ref):
    i = pl.program_id(0)

    @pl.when(i == 0)
    def _():
        pool_ref[...] = jnp.zeros_like(pool_ref)

    a = agg_ref[0] + agg_ref[1] + h_ref[...]
    dinv = 1.0 / (deg_ref[0] + deg_ref[1] + 1.0)
    h3 = a * dinv + b_ref[...]
    rows = i * BM + lax.broadcasted_iota(jnp.int32, (BM, 1), 0)
    h3 = jnp.where(rows < N_NODES, h3, 0.0)
    onehot = (bidx_ref[...] ==
              lax.broadcasted_iota(jnp.int32, (1, NUM_GRAPHS), 1)
              ).astype(jnp.float32)
    dn = (((0,), (0,)), ((), ()))
    pool_ref[...] += lax.dot_general(onehot, h3, dn,
                                     preferred_element_type=jnp.float32)

    @pl.when(i == pl.num_programs(0) - 1)
    def _():
        pooled = pool_ref[...] * cinv_ref[...]
        o_ref[...] = jnp.dot(pooled, wh_ref[...],
                             preferred_element_type=jnp.float32) + bh_ref[...]


def _tail(agg, h, deg, b, bidx, cinv, wh, bh):
    return pl.pallas_call(
        _tail_body,
        grid=(N_PAD // BM,),
        in_specs=[pl.BlockSpec((2, BM, C), lambda i: (0, i, 0)),
                  pl.BlockSpec((BM, C), lambda i: (i, 0)),
                  pl.BlockSpec((2, BM, 1), lambda i: (0, i, 0)),
                  pl.BlockSpec((1, C), lambda i: (0, 0)),
                  pl.BlockSpec((BM, 1), lambda i: (i, 0)),
                  pl.BlockSpec((NUM_GRAPHS, C), lambda i: (0, 0)),
                  pl.BlockSpec((C, C_OUT), lambda i: (0, 0)),
                  pl.BlockSpec((1, C_OUT), lambda i: (0, 0))],
        out_specs=pl.BlockSpec((NUM_GRAPHS, C_OUT), lambda i: (0, 0)),
        out_shape=jax.ShapeDtypeStruct((NUM_GRAPHS, C_OUT), jnp.float32),
        scratch_shapes=[pltpu.VMEM((NUM_GRAPHS, C), jnp.float32)],
    )(agg, h, deg, b, bidx, cinv, wh, bh)


def kernel(x, edge_index, batch_idx, W1, b1, W2, b2, Wh, bh):
    ei = edge_index.astype(jnp.int32)
    z2 = jnp.zeros((ROWS_PER_TILE, C), jnp.float32)
    z1 = jnp.zeros((ROWS_PER_TILE,), jnp.float32)
    bidx = jnp.concatenate(
        [batch_idx.astype(jnp.int32),
         jnp.full((N_PAD - N_NODES,), NUM_GRAPHS, jnp.int32)]
    ).reshape(N_PAD, 1)

    cinv = _cnt(bidx)
    h1 = _matmul(x, W1)
    agg1, deg = _mp_call(h1, ei, z2, z1)
    deg3 = deg.reshape(2, N_PAD, 1)
    h2 = _mid(agg1, h1, deg3, b1.reshape(1, C), W2)
    agg2 = _mp_call_nodeg(h2, ei, z2)
    out = _tail(agg2, h2, deg3, b2.reshape(1, C), bidx, cinv, Wh,
                bh.reshape(1, C_OUT))
    return out
